# baseline (speedup 1.0000x reference)
import functools

import jax
import jax.numpy as jnp
from jax import lax
from jax.experimental import pallas as pl
from jax.experimental.pallas import tpu as pltpu
from jax.experimental.pallas import tpu_sc as plsc

D_MODEL = 64
NUM_WORKERS = 32
BCHUNK = 128
LANES = 16


@functools.lru_cache(maxsize=None)
def _make_gather(batch: int, hist: int):
    assert batch == NUM_WORKERS * BCHUNK
    n_btiles = batch // BCHUNK
    mesh = plsc.VectorSubcoreMesh(core_axis_name="c", subcore_axis_name="s")

    @functools.partial(
        pl.kernel,
        mesh=mesh,
        out_type=jax.ShapeDtypeStruct(
            (hist, D_MODEL // 8, n_btiles, 8, BCHUNK), jnp.float32
        ),
        scratch_types=[
            pltpu.VMEM((BCHUNK,), jnp.int32),
            pltpu.VMEM((BCHUNK, D_MODEL), jnp.float32),
            pltpu.VMEM((D_MODEL // 8, 8, BCHUNK), jnp.float32),
            pltpu.SemaphoreType.DMA,
        ],
        compiler_params=pltpu.CompilerParams(
            use_tc_tiling_on_sc=False, needs_layout_passes=False
        ),
    )
    def gather_kernel(idx_hbm, table_hbm, out_hbm, idx_v, rows_v, tile_v, sem):
        wid = lax.axis_index("s") * 2 + lax.axis_index("c")
        lane = lax.broadcasted_iota(jnp.int32, (LANES,), 0)

        def body(t, carry):
            pltpu.sync_copy(
                idx_hbm.at[pl.ds(t * batch + wid * BCHUNK, BCHUNK)], idx_v
            )
            pltpu.async_copy(table_hbm.at[idx_v], rows_v, sem).wait()
            for c8 in range(D_MODEL // 8):
                for cc in range(8):
                    col = jnp.full((LANES,), 8 * c8 + cc, jnp.int32)
                    for b0 in range(BCHUNK // LANES):
                        v = plsc.load_gather(
                            rows_v, [lane + b0 * LANES, col]
                        )
                        tile_v[c8, cc, pl.ds(b0 * LANES, LANES)] = v
            pltpu.sync_copy(tile_v, out_hbm.at[t, :, wid])
            return carry

        lax.fori_loop(0, hist, body, 0)

    return gather_kernel


def kernel(x, table):
    b, h = x.shape
    idx_t = x.T.reshape(-1).astype(jnp.int32)
    out5 = _make_gather(b, h)(idx_t, table)
    return (
        out5.transpose(2, 4, 0, 1, 3).reshape(b, h, D_MODEL)
    )


# speedup vs baseline: 1.3122x; 1.3122x over previous
import functools

import jax
import jax.numpy as jnp
from jax import lax
from jax.experimental import pallas as pl
from jax.experimental.pallas import tpu as pltpu
from jax.experimental.pallas import tpu_sc as plsc

D_MODEL = 64
NUM_WORKERS = 32
BCHUNK = 128
LANES = 16


@functools.lru_cache(maxsize=None)
def _make_gather(batch: int, hist: int):
    assert batch == NUM_WORKERS * BCHUNK
    n_btiles = batch // BCHUNK
    mesh = plsc.VectorSubcoreMesh(core_axis_name="c", subcore_axis_name="s")

    @functools.partial(
        pl.kernel,
        mesh=mesh,
        out_type=jax.ShapeDtypeStruct(
            (hist, D_MODEL // 8, n_btiles, 8, BCHUNK), jnp.float32
        ),
        scratch_types=[
            pltpu.VMEM((BCHUNK,), jnp.int32),
            pltpu.VMEM((BCHUNK, D_MODEL), jnp.float32),
            pltpu.VMEM((D_MODEL // 8, 8, BCHUNK), jnp.float32),
            pltpu.SemaphoreType.DMA,
        ],
        compiler_params=pltpu.CompilerParams(
            use_tc_tiling_on_sc=False, needs_layout_passes=False
        ),
    )
    def gather_kernel(idx_hbm, table_hbm, out_hbm, idx_v, rows_v, tile_v, sem):
        wid = lax.axis_index("s") * 2 + lax.axis_index("c")
        lane = lax.broadcasted_iota(jnp.int32, (LANES,), 0)
        c8_idx = [(lane + 16 * k) // 8 for k in range(D_MODEL // LANES)]
        cc_idx = [(lane + 16 * k) % 8 for k in range(D_MODEL // LANES)]

        def body(t, carry):
            pltpu.sync_copy(
                idx_hbm.at[pl.ds(t * batch + wid * BCHUNK, BCHUNK)], idx_v
            )
            pltpu.async_copy(table_hbm.at[idx_v], rows_v, sem).wait()

            @plsc.parallel_loop(0, BCHUNK, unroll=8)
            def transpose_rows(b):
                bs = jnp.full((LANES,), 0, jnp.int32) + b
                for k in range(D_MODEL // LANES):
                    v = rows_v[b, pl.ds(k * LANES, LANES)]
                    plsc.store_scatter(tile_v, [c8_idx[k], cc_idx[k], bs], v)

            pltpu.sync_copy(tile_v, out_hbm.at[t, :, wid])
            return carry

        lax.fori_loop(0, hist, body, 0)

    return gather_kernel


def kernel(x, table):
    b, h = x.shape
    idx_t = x.T.reshape(-1).astype(jnp.int32)
    out5 = _make_gather(b, h)(idx_t, table)
    return (
        out5.transpose(2, 4, 0, 1, 3).reshape(b, h, D_MODEL)
    )


# speedup vs baseline: 1.6730x; 1.2750x over previous
import functools

import numpy as np

import jax
import jax.numpy as jnp
from jax import lax
from jax.experimental import pallas as pl
from jax.experimental.pallas import tpu as pltpu
from jax.experimental.pallas import tpu_sc as plsc

D_MODEL = 64
NUM_WORKERS = 32
BCHUNK = 128
LANES = 16

@functools.lru_cache(maxsize=None)
def _make_gather(batch: int, hist: int):
    assert batch == NUM_WORKERS * BCHUNK and hist % 2 == 0
    n_btiles = batch // BCHUNK
    mesh = plsc.VectorSubcoreMesh(core_axis_name="c", subcore_axis_name="s")

    @functools.partial(
        pl.kernel,
        mesh=mesh,
        out_type=jax.ShapeDtypeStruct(
            (hist, D_MODEL // 8, n_btiles, 8, BCHUNK), jnp.float32
        ),
        scratch_types=[
            pltpu.VMEM((hist, BCHUNK), jnp.int32),
            pltpu.VMEM((BCHUNK, D_MODEL), jnp.float32),
            pltpu.VMEM((BCHUNK, D_MODEL), jnp.float32),
            pltpu.VMEM((D_MODEL // 8, 8, BCHUNK), jnp.float32),
            pltpu.VMEM((D_MODEL // 8, 8, BCHUNK), jnp.float32),
            pltpu.SemaphoreType.DMA,
            pltpu.SemaphoreType.DMA,
            pltpu.SemaphoreType.DMA,
            pltpu.SemaphoreType.DMA,
        ],
        compiler_params=pltpu.CompilerParams(
            use_tc_tiling_on_sc=False, needs_layout_passes=False
        ),
    )
    def gather_kernel(
        idx_hbm, table_hbm, out_hbm,
        idx_v, rows0, rows1, tile0, tile1,
        sem_g0, sem_g1, sem_o0, sem_o1,
    ):
        wid = lax.axis_index("s") * 2 + lax.axis_index("c")

        lane = lax.broadcasted_iota(jnp.int32, (LANES,), 0)
        c8s = [(lane + LANES * k) // 8 for k in range(D_MODEL // LANES)]
        ccs = [(lane + LANES * k) % 8 for k in range(D_MODEL // LANES)]

        pltpu.sync_copy(idx_hbm.at[:, pl.ds(wid * BCHUNK, BCHUNK)], idx_v)

        def transpose(rows, tile):
            @plsc.parallel_loop(0, BCHUNK, unroll=8)
            def _(b):
                bs = jnp.full((LANES,), 0, jnp.int32) + b
                for k in range(D_MODEL // LANES):
                    v = rows[b, pl.ds(k * LANES, LANES)]
                    plsc.store_scatter(tile, [c8s[k], ccs[k], bs], v)

        def gather(t, rows, sem):
            pltpu.async_copy(table_hbm.at[idx_v.at[t]], rows, sem)

        def gather_wait(t, rows, sem):
            pltpu.make_async_copy(table_hbm.at[idx_v.at[t]], rows, sem).wait()

        def drain_out(tile, sem):
            pltpu.make_async_copy(out_hbm.at[0, :, wid], tile, sem).wait()

        gather(0, rows0, sem_g0)

        def body(i, carry):
            t0 = 2 * i
            t1 = t0 + 1
            gather(t1, rows1, sem_g1)
            gather_wait(t0, rows0, sem_g0)

            @pl.when(i > 0)
            def _():
                drain_out(tile0, sem_o0)

            transpose(rows0, tile0)
            pltpu.async_copy(tile0, out_hbm.at[t0, :, wid], sem_o0)

            @pl.when(t0 + 2 < hist)
            def _():
                gather(t0 + 2, rows0, sem_g0)

            gather_wait(t1, rows1, sem_g1)

            @pl.when(i > 0)
            def _():
                drain_out(tile1, sem_o1)

            transpose(rows1, tile1)
            pltpu.async_copy(tile1, out_hbm.at[t1, :, wid], sem_o1)
            return carry

        lax.fori_loop(0, hist // 2, body, 0)
        drain_out(tile0, sem_o0)
        drain_out(tile1, sem_o1)

    return gather_kernel


def kernel(x, table):
    b, h = x.shape
    idx_t = x.T.astype(jnp.int32)
    out5 = _make_gather(b, h)(idx_t, table)
    return out5.transpose(2, 4, 0, 1, 3).reshape(b, h, D_MODEL)
